# baseline (speedup 1.0000x reference)
import jax
import jax.numpy as jnp
from jax.experimental import pallas as pl
from jax.experimental.pallas import tpu as pltpu

N = 1024
IN = 128
OUT = 128
B = 256
NB = N // B
NEG_LOG2E = -1.4426950408889634


def _dot(a, b, dims):
    return jax.lax.dot_general(a, b, (dims, ((), ())),
                               preferred_element_type=jnp.float32)


def _body(x_ref, aa_ref, ab_ref, wg_ref, bg_ref, wn_ref, an_ref,
          wa_ref, ba_ref, wb_ref, bb_ref, out_ref,
          dense_s, xh_s, xg_s, pd_s, va_s, vb_s, r_s, ga_s, m1_s):
    bf = jnp.bfloat16
    p = pl.program_id(0)
    i = pl.program_id(1)
    rows = pl.ds(i * B, B)

    @pl.when(jnp.logical_and(p == 0, i == 0))
    def _init():
        x = x_ref[...]
        xh = _dot(x, wn_ref[...], ((1,), (0,)))
        xh_s[...] = xh
        xg_s[...] = _dot(x, wg_ref[...], ((1,), (0,))).astype(bf)
        an = an_ref[...]
        pd_s[...] = _dot(an[:, OUT:], xh, ((1,), (1,))) * NEG_LOG2E
        va_s[...] = _dot(wa_ref[:, :IN], x, ((1,), (1,)))
        vb_s[...] = _dot(wb_ref[:, :IN], x, ((1,), (1,)))

    @pl.when(p == 0)
    def _phase0():
        aa = aa_ref[...]
        x_blk = x_ref[rows, :]
        xh_blk = xh_s[rows, :]
        an = an_ref[...]
        ps = _dot(xh_blk, an[:, :OUT], ((1,), (1,))) * NEG_LOG2E
        pm = ps + pd_s[...]
        e = jnp.exp2(jnp.minimum(pm, 0.01 * pm))
        d = aa * e
        dense_s[rows, :] = d.astype(bf)
        r_s[rows, :] = jnp.sum(d, axis=1, keepdims=True)
        m_a = jnp.sum(aa * va_s[...], axis=1, keepdims=True)
        u_a = _dot(x_blk, wa_ref[:, IN:], ((1,), (1,)))
        ga_s[rows, :] = jax.nn.sigmoid(m_a + u_a + ba_ref[0])

    @pl.when(jnp.logical_and(p == 1, i == 0))
    def _mk_m1():
        inv = 1.0 / (r_s[...] + 1e-05)
        m1_s[...] = (xh_s[...] * inv).astype(bf)

    @pl.when(p == 1)
    def _phase1():
        ab = ab_ref[...]
        x_blk = x_ref[rows, :]
        x_a = _dot(dense_s[rows, :], m1_s[...], ((1,), (0,)))
        xbb = _dot(ab.astype(bf), xg_s[...], ((1,), (0,))) + bg_ref[...]
        m_b = jnp.sum(ab * vb_s[...], axis=1, keepdims=True)
        u_b = _dot(x_blk, wb_ref[:, IN:], ((1,), (1,)))
        gate_b = jax.nn.sigmoid(m_b + u_b + bb_ref[0])
        out_ref[...] = jax.nn.sigmoid(ga_s[rows, :] * x_a + gate_b * xbb)


@jax.jit
def kernel(x, adj_a, adj_b, W_gcn, b_gcn, W_na, a_na, Wa, ba, Wb, bb):
    f32 = jnp.float32
    bf = jnp.bfloat16
    grid = (2, NB)

    full = lambda shape: pl.BlockSpec(shape, lambda p, i: (0, 0))
    return pl.pallas_call(
        _body,
        grid=grid,
        in_specs=[
            full((N, IN)),
            pl.BlockSpec((B, N), lambda p, i: (jnp.where(p == 0, i, NB - 1), 0)),
            pl.BlockSpec((B, N), lambda p, i: (jnp.where(p == 0, 0, i), 0)),
            full((IN, OUT)),
            full((1, OUT)),
            full((IN, OUT)),
            full((1, 2 * OUT)),
            full((1, 2 * IN)),
            pl.BlockSpec(memory_space=pltpu.SMEM),
            full((1, 2 * IN)),
            pl.BlockSpec(memory_space=pltpu.SMEM),
        ],
        out_specs=pl.BlockSpec((B, OUT), lambda p, i: (jnp.where(p == 0, 0, i), 0)),
        out_shape=jax.ShapeDtypeStruct((N, OUT), f32),
        scratch_shapes=[
            pltpu.VMEM((N, N), bf),
            pltpu.VMEM((N, OUT), f32),
            pltpu.VMEM((N, OUT), bf),
            pltpu.VMEM((1, N), f32),
            pltpu.VMEM((1, N), f32),
            pltpu.VMEM((1, N), f32),
            pltpu.VMEM((N, 1), f32),
            pltpu.VMEM((N, 1), f32),
            pltpu.VMEM((N, OUT), bf),
        ],
    )(x, adj_a, adj_b, W_gcn, b_gcn.reshape(1, OUT), W_na, a_na,
      Wa, ba, Wb, bb)


# speedup vs baseline: 1.2285x; 1.2285x over previous
import jax
import jax.numpy as jnp
from jax.experimental import pallas as pl
from jax.experimental.pallas import tpu as pltpu

N = 1024
IN = 128
OUT = 128
B = 256
NB = N // B
NEG_LOG2E = -1.4426950408889634


def _dot(a, b, dims):
    return jax.lax.dot_general(a, b, (dims, ((), ())),
                               preferred_element_type=jnp.float32)


def _body(x_ref, aa_hbm, ab_hbm, wg_ref, bg_ref, wn_ref, an_ref,
          wa_ref, ba_ref, wb_ref, bb_ref, out_ref,
          aa_s, ab_s, dense_s, sem_a, sem_b):
    bf = jnp.bfloat16

    cps_a = [pltpu.make_async_copy(aa_hbm.at[pl.ds(k * B, B), :],
                                   aa_s.at[pl.ds(k * B, B), :],
                                   sem_a.at[k]) for k in range(NB)]
    cps_b = [pltpu.make_async_copy(ab_hbm.at[pl.ds(k * B, B), :],
                                   ab_s.at[pl.ds(k * B, B), :],
                                   sem_b.at[k]) for k in range(NB)]
    for c in cps_a:
        c.start()
    for c in cps_b:
        c.start()

    x = x_ref[...]
    xh = _dot(x, wn_ref[...], ((1,), (0,)))
    xg = _dot(x, wg_ref[...], ((1,), (0,))).astype(bf)
    an = an_ref[...]
    ps = _dot(xh, an[:, :OUT], ((1,), (1,))) * NEG_LOG2E
    pd = _dot(an[:, OUT:], xh, ((1,), (1,))) * NEG_LOG2E
    va = _dot(wa_ref[:, :IN], x, ((1,), (1,)))
    vb = _dot(wb_ref[:, :IN], x, ((1,), (1,)))
    u_a = _dot(x, wa_ref[:, IN:], ((1,), (1,)))
    u_b = _dot(x, wb_ref[:, IN:], ((1,), (1,)))

    r_parts, ma_parts = [], []
    for k in range(NB):
        rows = pl.ds(k * B, B)
        cps_a[k].wait()
        aa = aa_s[rows, :]
        pm = ps[k * B:(k + 1) * B, :] + pd
        e = jnp.exp2(jnp.minimum(pm, 0.01 * pm))
        d = aa * e
        dense_s[rows, :] = d.astype(bf)
        r_parts.append(jnp.sum(d, axis=1, keepdims=True))
        ma_parts.append(jnp.sum(aa * va, axis=1, keepdims=True))

    r = jnp.concatenate(r_parts, axis=0)
    m_a = jnp.concatenate(ma_parts, axis=0)
    gate_a = jax.nn.sigmoid(m_a + u_a + ba_ref[0])
    m1 = (xh * (1.0 / (r + 1e-05))).astype(bf)

    for k in range(NB):
        rows = pl.ds(k * B, B)
        cps_b[k].wait()
        ab = ab_s[rows, :]
        x_a = _dot(dense_s[rows, :], m1, ((1,), (0,)))
        xbb = _dot(ab.astype(bf), xg, ((1,), (0,))) + bg_ref[...]
        m_b = jnp.sum(ab * vb, axis=1, keepdims=True)
        gate_b = jax.nn.sigmoid(m_b + u_b[k * B:(k + 1) * B, :] + bb_ref[0])
        out_ref[rows, :] = jax.nn.sigmoid(
            gate_a[k * B:(k + 1) * B, :] * x_a + gate_b * xbb)


@jax.jit
def kernel(x, adj_a, adj_b, W_gcn, b_gcn, W_na, a_na, Wa, ba, Wb, bb):
    f32 = jnp.float32
    bf = jnp.bfloat16

    vmem = lambda: pl.BlockSpec(memory_space=pltpu.MemorySpace.VMEM)
    return pl.pallas_call(
        _body,
        in_specs=[
            vmem(),
            pl.BlockSpec(memory_space=pltpu.MemorySpace.HBM),
            pl.BlockSpec(memory_space=pltpu.MemorySpace.HBM),
            vmem(),
            vmem(),
            vmem(),
            vmem(),
            vmem(),
            pl.BlockSpec(memory_space=pltpu.MemorySpace.SMEM),
            vmem(),
            pl.BlockSpec(memory_space=pltpu.MemorySpace.SMEM),
        ],
        out_specs=vmem(),
        out_shape=jax.ShapeDtypeStruct((N, OUT), f32),
        scratch_shapes=[
            pltpu.VMEM((N, N), f32),
            pltpu.VMEM((N, N), f32),
            pltpu.VMEM((N, N), bf),
            pltpu.SemaphoreType.DMA((NB,)),
            pltpu.SemaphoreType.DMA((NB,)),
        ],
    )(x, adj_a, adj_b, W_gcn, b_gcn.reshape(1, OUT), W_na, a_na,
      Wa, ba, Wb, bb)


# speedup vs baseline: 1.3831x; 1.1259x over previous
import jax
import jax.numpy as jnp
from jax.experimental import pallas as pl
from jax.experimental.pallas import tpu as pltpu

N = 1024
IN = 128
OUT = 128
B = 256
NB = N // B
NEG_LOG2E = -1.4426950408889634


def _dot(a, b, dims):
    return jax.lax.dot_general(a, b, (dims, ((), ())),
                               preferred_element_type=jnp.float32)


def _body(x_ref, aa_hbm, ab_hbm, wg_ref, bg_ref, wn_ref, an_ref,
          wa_ref, ba_ref, wb_ref, bb_ref, out_ref,
          aa_s, ab_s, dense_s, xbb_s, ga_s, sem_a, sem_b):
    bf = jnp.bfloat16

    cps_a = [pltpu.make_async_copy(aa_hbm.at[pl.ds(k * B, B), :],
                                   aa_s.at[pl.ds(k * B, B), :],
                                   sem_a.at[k]) for k in range(NB)]
    cps_b = [pltpu.make_async_copy(ab_hbm.at[pl.ds(k * B, B), :],
                                   ab_s.at[pl.ds(k * B, B), :],
                                   sem_b.at[k]) for k in range(NB)]
    for k in range(NB):
        cps_a[k].start()
        cps_b[k].start()

    x = x_ref[...]
    xh = _dot(x, wn_ref[...], ((1,), (0,)))
    xg = _dot(x, wg_ref[...], ((1,), (0,))).astype(bf)
    an = an_ref[...]
    ps = _dot(xh, an[:, :OUT], ((1,), (1,))) * NEG_LOG2E
    pd = _dot(an[:, OUT:], xh, ((1,), (1,))) * NEG_LOG2E
    va = _dot(wa_ref[:, :IN], x, ((1,), (1,)))
    vb = _dot(wb_ref[:, :IN], x, ((1,), (1,)))
    u_a = _dot(x, wa_ref[:, IN:], ((1,), (1,)))
    u_b = _dot(x, wb_ref[:, IN:], ((1,), (1,)))

    r_parts = []
    for k in range(NB):
        rows = pl.ds(k * B, B)
        sl = slice(k * B, (k + 1) * B)

        cps_a[k].wait()
        aa = aa_s[rows, :]
        pm = ps[sl, :] + pd
        e = jnp.exp2(jnp.minimum(pm, 0.01 * pm))
        d = aa * e
        dense_s[rows, :] = d.astype(bf)
        r_parts.append(jnp.sum(d, axis=1, keepdims=True))
        m_a = jnp.sum(aa * va, axis=1, keepdims=True)
        ga_s[rows, :] = jax.nn.sigmoid(m_a + u_a[sl, :] + ba_ref[0])

        cps_b[k].wait()
        ab = ab_s[rows, :]
        xbb = _dot(ab.astype(bf), xg, ((1,), (0,))) + bg_ref[...]
        m_b = jnp.sum(ab * vb, axis=1, keepdims=True)
        gate_b = jax.nn.sigmoid(m_b + u_b[sl, :] + bb_ref[0])
        xbb_s[rows, :] = gate_b * xbb

    r = jnp.concatenate(r_parts, axis=0)
    m1 = (xh * (1.0 / (r + 1e-05))).astype(bf)

    for k in range(NB):
        rows = pl.ds(k * B, B)
        x_a = _dot(dense_s[rows, :], m1, ((1,), (0,)))
        out_ref[rows, :] = jax.nn.sigmoid(ga_s[rows, :] * x_a + xbb_s[rows, :])


@jax.jit
def kernel(x, adj_a, adj_b, W_gcn, b_gcn, W_na, a_na, Wa, ba, Wb, bb):
    f32 = jnp.float32
    bf = jnp.bfloat16

    vmem = lambda: pl.BlockSpec(memory_space=pltpu.MemorySpace.VMEM)
    return pl.pallas_call(
        _body,
        in_specs=[
            vmem(),
            pl.BlockSpec(memory_space=pltpu.MemorySpace.HBM),
            pl.BlockSpec(memory_space=pltpu.MemorySpace.HBM),
            vmem(),
            vmem(),
            vmem(),
            vmem(),
            vmem(),
            pl.BlockSpec(memory_space=pltpu.MemorySpace.SMEM),
            vmem(),
            pl.BlockSpec(memory_space=pltpu.MemorySpace.SMEM),
        ],
        out_specs=vmem(),
        out_shape=jax.ShapeDtypeStruct((N, OUT), f32),
        scratch_shapes=[
            pltpu.VMEM((N, N), f32),
            pltpu.VMEM((N, N), f32),
            pltpu.VMEM((N, N), bf),
            pltpu.VMEM((N, OUT), f32),
            pltpu.VMEM((N, 1), f32),
            pltpu.SemaphoreType.DMA((NB,)),
            pltpu.SemaphoreType.DMA((NB,)),
        ],
    )(x, adj_a, adj_b, W_gcn, b_gcn.reshape(1, OUT), W_na, a_na,
      Wa, ba, Wb, bb)
